# baseline (speedup 1.0000x reference)
import functools

import jax
import jax.numpy as jnp
from jax import lax
from jax.experimental import pallas as pl
from jax.experimental.pallas import tpu as pltpu
from jax.experimental.pallas import tpu_sc as plsc

B = 16384
L = 200
VOCAB = 7800
DIM = 64
VPAD = 7808
NC, NS = 2, 16
NW = NC * NS
RPW = B // NW
GROUPS = RPW // 16
TR = L // 8


def _project_body(e_ref, w_ref, o_ref):
    o_ref[...] = jnp.sum(e_ref[...] * w_ref[...], axis=1, keepdims=True)


def _project(table_pad, lin_w):
    return pl.pallas_call(
        _project_body,
        out_shape=jax.ShapeDtypeStruct((VPAD, 1), jnp.float32),
    )(table_pad, lin_w)


_MESH = plsc.VectorSubcoreMesh(core_axis_name="c", subcore_axis_name="s")


@functools.partial(
    pl.kernel,
    out_type=jax.ShapeDtypeStruct((B,), jnp.float32),
    mesh=_MESH,
    compiler_params=pltpu.CompilerParams(needs_layout_passes=False),
    scratch_types=[
        pltpu.VMEM((VPAD,), jnp.float32),
        pltpu.VMEM((8 * TR, 128), jnp.int32),
        pltpu.VMEM((8 * TR, 128), jnp.int32),
        pltpu.VMEM((RPW,), jnp.float32),
        pltpu.VMEM((16,), jnp.float32),
        pltpu.VMEM((RPW,), jnp.float32),
        pltpu.SemaphoreType.DMA,
        pltpu.SemaphoreType.DMA,
    ],
)
def _sc_pool(s_hbm, x_hbm, len_hbm, bias_hbm, out_hbm,
             s_v, xb0, xb1, len_v, bias_v, out_v, sem0, sem1):
    wid = lax.axis_index("s") * NC + lax.axis_index("c")
    base = wid * RPW
    bufs = (xb0, xb1)
    sems = (sem0, sem1)

    def start(tc):
        buf, sem = bufs[tc % 2], sems[tc % 2]
        return [
            pltpu.async_copy(
                x_hbm.at[pl.ds(tr * 1024 + 32 * wid + 8 * tc, 8), :],
                buf.at[pl.ds(tr * 8, 8), :], sem)
            for tr in range(TR)
        ]

    cps = start(0)
    pltpu.sync_copy(s_hbm, s_v)
    pltpu.sync_copy(len_hbm.at[pl.ds(base, RPW)], len_v)
    pltpu.sync_copy(bias_hbm, bias_v)

    lanes = lax.iota(jnp.int32, 16)
    bias = bias_v[...]
    zero = jnp.zeros((16,), jnp.float32)
    zero_i = jnp.zeros((16,), jnp.int32)

    for tc in range(4):
        for cp in cps:
            cp.wait()
        if tc + 1 < 4:
            cps = start(tc + 1)
        x_v = bufs[tc % 2]

        def group_body(j, carry):
            cvec = j * 16 + lanes

            def band(tr, st):
                acc0, acc1, i0 = st
                for rr in range(40):
                    xv = plsc.load_gather(x_v, [i0 + rr, cvec])
                    val = plsc.load_gather(s_v, [xv])
                    if rr % 2 == 0:
                        acc0 = acc0 + val
                    else:
                        acc1 = acc1 + val
                return acc0, acc1, i0 + 40

            a0, a1, _ = lax.fori_loop(0, TR // 5, band, (zero, zero, zero_i))
            off = tc * 128 + j * 16
            z = (a0 + a1) / len_v[pl.ds(off, 16)] + bias
            out_v[pl.ds(off, 16)] = 1.0 / (1.0 + jnp.exp(-z))
            return carry

        lax.fori_loop(0, 8, group_body, 0)
    pltpu.sync_copy(out_v, out_hbm.at[pl.ds(base, RPW)])


def kernel(x, length, embed_table, lin_w, lin_b):
    table_pad = jnp.pad(embed_table, ((0, VPAD - VOCAB), (0, 0)))
    s = _project(table_pad, lin_w).reshape(VPAD)
    bias16 = jnp.broadcast_to(lin_b, (16,)).astype(jnp.float32)
    xt = (x.T.reshape(TR, 8, B // 128, 128)
          .transpose(0, 2, 1, 3)
          .reshape(TR * (B // 128) * 8, 128))
    y = _sc_pool(s, xt, length, bias16)
    return y.reshape(B, 1)


# speedup vs baseline: 1.1044x; 1.1044x over previous
import functools

import jax
import jax.numpy as jnp
from jax import lax
from jax.experimental import pallas as pl
from jax.experimental.pallas import tpu as pltpu
from jax.experimental.pallas import tpu_sc as plsc

B = 16384
L = 200
VOCAB = 7800
DIM = 64
VPAD = 7808
NC, NS = 2, 16
NW = NC * NS
RPW = B // NW
GROUPS = RPW // 16
TR = L // 8


def _project_body(e_ref, w_ref, o_ref):
    o_ref[...] = jnp.sum(e_ref[...] * w_ref[...], axis=1, keepdims=True)


def _project(table_pad, lin_w):
    return pl.pallas_call(
        _project_body,
        out_shape=jax.ShapeDtypeStruct((VPAD, 1), jnp.float32),
    )(table_pad, lin_w)


_MESH = plsc.VectorSubcoreMesh(core_axis_name="c", subcore_axis_name="s")


@functools.partial(
    pl.kernel,
    out_type=jax.ShapeDtypeStruct((B,), jnp.float32),
    mesh=_MESH,
    compiler_params=pltpu.CompilerParams(needs_layout_passes=False),
    scratch_types=[
        pltpu.VMEM((VPAD,), jnp.float32),
        pltpu.VMEM((8 * TR, 128), jnp.int32),
        pltpu.VMEM((8 * TR, 128), jnp.int32),
        pltpu.VMEM((RPW,), jnp.float32),
        pltpu.VMEM((16,), jnp.float32),
        pltpu.VMEM((RPW,), jnp.float32),
        pltpu.SemaphoreType.DMA,
        pltpu.SemaphoreType.DMA,
    ],
)
def _sc_pool(s_hbm, x_hbm, len_hbm, bias_hbm, out_hbm,
             s_v, xb0, xb1, len_v, bias_v, out_v, sem0, sem1):
    wid = lax.axis_index("s") * NC + lax.axis_index("c")
    base = wid * RPW
    bufs = (xb0, xb1)
    sems = (sem0, sem1)

    def start(tc):
        buf, sem = bufs[tc % 2], sems[tc % 2]
        return [
            pltpu.async_copy(
                x_hbm.at[pl.ds(tr * 1024 + 32 * wid + 8 * tc, 8), :],
                buf.at[pl.ds(tr * 8, 8), :], sem)
            for tr in range(TR)
        ]

    cps = start(0)
    pltpu.sync_copy(s_hbm, s_v)
    pltpu.sync_copy(len_hbm.at[pl.ds(base, RPW)], len_v)
    pltpu.sync_copy(bias_hbm, bias_v)

    lanes = lax.iota(jnp.int32, 16)
    bias = bias_v[...]
    zero = jnp.zeros((16,), jnp.float32)
    zero_i = jnp.zeros((16,), jnp.int32)

    for tc in range(4):
        for cp in cps:
            cp.wait()
        if tc + 1 < 4:
            cps = start(tc + 1)
        x_v = bufs[tc % 2]

        def group_body(j, carry):
            cvec = j * 16 + lanes

            def band(tr, st):
                acc0, acc1, i0 = st
                for rr in range(16):
                    xv = plsc.load_gather(x_v, [i0 + rr, cvec])
                    val = plsc.load_gather(s_v, [xv])
                    if rr % 2 == 0:
                        acc0 = acc0 + val
                    else:
                        acc1 = acc1 + val
                return acc0, acc1, i0 + 16

            a0, a1, i0f = lax.fori_loop(0, 12, band, (zero, zero, zero_i))
            for rr in range(8):
                xv = plsc.load_gather(x_v, [i0f + rr, cvec])
                val = plsc.load_gather(s_v, [xv])
                if rr % 2 == 0:
                    a0 = a0 + val
                else:
                    a1 = a1 + val
            off = tc * 128 + j * 16
            z = (a0 + a1) / len_v[pl.ds(off, 16)] + bias
            out_v[pl.ds(off, 16)] = 1.0 / (1.0 + jnp.exp(-z))
            return carry

        lax.fori_loop(0, 8, group_body, 0)
    pltpu.sync_copy(out_v, out_hbm.at[pl.ds(base, RPW)])


def kernel(x, length, embed_table, lin_w, lin_b):
    table_pad = jnp.pad(embed_table, ((0, VPAD - VOCAB), (0, 0)))
    s = _project(table_pad, lin_w).reshape(VPAD)
    bias16 = jnp.broadcast_to(lin_b, (16,)).astype(jnp.float32)
    xt = (x.T.reshape(TR, 8, B // 128, 128)
          .transpose(0, 2, 1, 3)
          .reshape(TR * (B // 128) * 8, 128))
    y = _sc_pool(s, xt, length, bias16)
    return y.reshape(B, 1)


# speedup vs baseline: 1.3104x; 1.1865x over previous
import functools

import jax
import jax.numpy as jnp
from jax import lax
from jax.experimental import pallas as pl
from jax.experimental.pallas import tpu as pltpu
from jax.experimental.pallas import tpu_sc as plsc

B = 16384
L = 200
VOCAB = 7800
DIM = 64
VPAD = 7808
NC, NS = 2, 16
NW = NC * NS
RPW = B // NW
GROUPS = RPW // 16
TR = L // 8


def _project_body(et_ref, w_ref, o_ref):
    o_ref[...] = jnp.sum(et_ref[...] * w_ref[...].T, axis=0, keepdims=True)


def _project(table_t_pad, lin_w):
    return pl.pallas_call(
        _project_body,
        out_shape=jax.ShapeDtypeStruct((1, VPAD), jnp.float32),
    )(table_t_pad, lin_w)


_MESH = plsc.VectorSubcoreMesh(core_axis_name="c", subcore_axis_name="s")


@functools.partial(
    pl.kernel,
    out_type=jax.ShapeDtypeStruct((B,), jnp.float32),
    mesh=_MESH,
    compiler_params=pltpu.CompilerParams(needs_layout_passes=False),
    scratch_types=[
        pltpu.VMEM((VPAD,), jnp.float32),
        pltpu.VMEM((8 * TR, 128), jnp.int32),
        pltpu.VMEM((8 * TR, 128), jnp.int32),
        pltpu.VMEM((RPW,), jnp.float32),
        pltpu.VMEM((16,), jnp.float32),
        pltpu.VMEM((RPW,), jnp.float32),
        pltpu.SemaphoreType.DMA,
        pltpu.SemaphoreType.DMA,
    ],
)
def _sc_pool(s_hbm, x_hbm, len_hbm, bias_hbm, out_hbm,
             s_v, xb0, xb1, len_v, bias_v, out_v, sem0, sem1):
    wid = lax.axis_index("s") * NC + lax.axis_index("c")
    base = wid * RPW
    bufs = (xb0, xb1)
    sems = (sem0, sem1)

    def start(tc):
        buf, sem = bufs[tc % 2], sems[tc % 2]
        return [
            pltpu.async_copy(
                x_hbm.at[pl.ds(tr * 1024 + 32 * wid + 8 * tc, 8), :],
                buf.at[pl.ds(tr * 8, 8), :], sem)
            for tr in range(TR)
        ]

    cps = start(0)
    pltpu.sync_copy(s_hbm, s_v)
    pltpu.sync_copy(len_hbm.at[pl.ds(base, RPW)], len_v)
    pltpu.sync_copy(bias_hbm, bias_v)

    lanes = lax.iota(jnp.int32, 16)
    bias = bias_v[...]
    zero = jnp.zeros((16,), jnp.float32)
    zero_i = jnp.zeros((16,), jnp.int32)

    for tc in range(4):
        for cp in cps:
            cp.wait()
        if tc + 1 < 4:
            cps = start(tc + 1)
        x_v = bufs[tc % 2]

        def group_body(j, carry):
            cvec = j * 16 + lanes

            def band(tr, st):
                acc0, acc1, i0 = st
                for r in range(8):
                    xv = plsc.load_gather(x_v, [i0 + r, cvec])
                    val = plsc.load_gather(s_v, [xv])
                    if r % 2 == 0:
                        acc0 = acc0 + val
                    else:
                        acc1 = acc1 + val
                return acc0, acc1, i0 + 8

            a0, a1, _ = lax.fori_loop(0, TR, band, (zero, zero, zero_i))
            off = tc * 128 + j * 16
            z = (a0 + a1) / len_v[pl.ds(off, 16)] + bias
            out_v[pl.ds(off, 16)] = 1.0 / (1.0 + jnp.exp(-z))
            return carry

        lax.fori_loop(0, 8, group_body, 0)
    pltpu.sync_copy(out_v, out_hbm.at[pl.ds(base, RPW)])


def kernel(x, length, embed_table, lin_w, lin_b):
    table_t_pad = jnp.pad(embed_table, ((0, VPAD - VOCAB), (0, 0))).T
    s = _project(table_t_pad, lin_w).reshape(VPAD)
    bias16 = jnp.broadcast_to(lin_b, (16,)).astype(jnp.float32)
    xt = (x.T.reshape(TR, 8, B // 128, 128)
          .transpose(0, 2, 1, 3)
          .reshape(TR * (B // 128) * 8, 128))
    y = _sc_pool(s, xt, length, bias16)
    return y.reshape(B, 1)
